# baseline (speedup 1.0000x reference)
import dataclasses
import functools

import jax
import jax.numpy as jnp
from jax import lax
from jax.experimental import pallas as pl
from jax.experimental.pallas import tpu as pltpu
from jax.experimental.pallas import tpu_sc as plsc

N = 10000
FE = 128
NC = 2
NS = 16
NW = NC * NS
L = 16
CH = 128
CPT = 80
KB = 4
EPT = CH * CPT
E_PAD = EPT * NW
N_PAD = 10112
RPT = N_PAD // NS
ZR = 79

_mesh = plsc.VectorSubcoreMesh(core_axis_name="c", subcore_axis_name="s")

_sc_params = pltpu.CompilerParams()
if "needs_layout_passes" in pltpu.CompilerParams.__dataclass_fields__:
    _sc_params = dataclasses.replace(_sc_params, needs_layout_passes=False)


@functools.partial(
    pl.kernel,
    out_type=jax.ShapeDtypeStruct((NW, N_PAD), jnp.float32),
    mesh=_mesh,
    scratch_types=[
        pltpu.VMEM((N_PAD,), jnp.float32),
        pltpu.VMEM((CPT, CH), jnp.int32),
    ],
    compiler_params=_sc_params,
)
def _sc_degree(dst_hbm, out_hbm, acc_v, idx_v):
    c = lax.axis_index("c")
    s = lax.axis_index("s")
    wid = s * NC + c
    z16 = jnp.zeros((L,), jnp.float32)

    @pl.loop(0, N_PAD, step=L)
    def _(i):
        acc_v[pl.ds(i, L)] = z16

    pltpu.sync_copy(dst_hbm.at[pl.ds(wid * CPT, CPT)], idx_v)
    ones = jnp.ones((L,), jnp.float32)

    @pl.loop(0, CPT)
    def _(r):
        @pl.loop(0, CH, step=L)
        def _(i):
            plsc.addupdate_scatter(acc_v, [idx_v[r, pl.ds(i, L)]], ones)

    pltpu.sync_copy(acc_v, out_hbm.at[wid])


@functools.partial(
    pl.kernel,
    out_type=jax.ShapeDtypeStruct((NC, N_PAD, FE), jnp.float32),
    mesh=_mesh,
    scratch_types=[
        pltpu.VMEM((2, 2, CH), jnp.int32),
        pltpu.VMEM((2, CH, FE), jnp.float32),
        pltpu.VMEM_SHARED((N_PAD, FE), jnp.float32),
        [pltpu.SemaphoreType.DMA] * 2,
        [pltpu.SemaphoreType.DMA] * 2,
    ],
)
def _sc_round(eidx_hbm, g_hbm, out_hbm, idxb, rows, acc_sh, isems, gsems):
    c = lax.axis_index("c")
    s = lax.axis_index("s")
    wid = s * NC + c
    base = wid * CPT
    z16 = jnp.zeros((L,), jnp.float32)

    @pl.loop(0, CH)
    def _(r):
        @pl.loop(0, FE, step=L)
        def _(f):
            rows[0, r, pl.ds(f, L)] = z16

    for k in range(RPT // CH):
        pltpu.sync_copy(rows.at[0], acc_sh.at[pl.ds(s * RPT + k * CH, CH)])
    if RPT % CH:
        pltpu.sync_copy(rows.at[0, pl.ds(0, RPT % CH)],
                        acc_sh.at[pl.ds(s * RPT + (RPT // CH) * CH,
                                        RPT % CH)])

    plsc.subcore_barrier()

    @pl.loop(0, CPT)
    def _(cc):
        pltpu.sync_copy(eidx_hbm.at[base + cc], idxb.at[0])
        pltpu.sync_copy(g_hbm.at[idxb.at[0, 0]], rows.at[0])
        pltpu.sync_copy(rows.at[0], acc_sh.at[idxb.at[0, 1]], add=True)

    plsc.subcore_barrier()
    pltpu.sync_copy(acc_sh.at[pl.ds(s * RPT, RPT)],
                    out_hbm.at[c, pl.ds(s * RPT, RPT)])


def _tc_proj_body(deg_ref, x_ref, win_ref, bin_ref, isd_ref, g_ref):
    deg = jnp.sum(deg_ref[...][:, :N], axis=0)
    isd = lax.rsqrt(deg + 1.0)
    isd_ref[...] = isd[:, None]
    h = jnp.maximum(
        jnp.dot(x_ref[...], win_ref[...], preferred_element_type=jnp.float32)
        + bin_ref[...], 0.0)
    g_ref[...] = h * isd[:, None]


def _tc_round_body(s_ref, isd_ref, w_ref, b_ref, g_ref):
    isd = isd_ref[...]
    agg = (s_ref[0, :N, :] + s_ref[1, :N, :]) * isd
    h = jnp.maximum(
        jnp.dot(agg, w_ref[...], preferred_element_type=jnp.float32)
        + b_ref[...], 0.0)
    g_ref[...] = h * isd


def _tc_final_body(s_ref, isd_ref, wp_ref, bp_ref, dt_ref, wt1_ref, bt1_ref,
                   wt2_ref, bt2_ref, w1a_ref, w1b_ref, b1_ref, w2_ref, b2_ref,
                   mu_ref, ls_ref, lam_ref, h_ref):
    isd = isd_ref[...]
    agg = (s_ref[0, :N, :] + s_ref[1, :N, :]) * isd
    hl = jnp.maximum(
        jnp.dot(agg, wp_ref[...], preferred_element_type=jnp.float32)
        + bp_ref[...], 0.0)
    h_ref[...] = hl
    e = jnp.maximum(dt_ref[...] * wt1_ref[...] + bt1_ref[...], 0.0)
    me = jnp.mean(e, axis=0, keepdims=True)
    ht = jnp.tanh(
        jnp.dot(me, wt2_ref[...], preferred_element_type=jnp.float32)
        + bt2_ref[...])
    const = jnp.dot(ht, w1b_ref[...], preferred_element_type=jnp.float32) \
        + b1_ref[...]
    hidden = jnp.maximum(
        jnp.dot(hl, w1a_ref[...], preferred_element_type=jnp.float32)
        + const, 0.0)
    out = jnp.dot(hidden, w2_ref[...], preferred_element_type=jnp.float32) \
        + b2_ref[...]
    mu = out[:, 0:1]
    ls2 = out[:, 1:2]
    mu_ref[...] = mu
    ls_ref[...] = ls2
    lam_ref[...] = jnp.exp(mu + 0.5 * jnp.exp(2.0 * ls2))


_tc_proj = pl.pallas_call(
    _tc_proj_body,
    out_shape=[
        jax.ShapeDtypeStruct((N, 1), jnp.float32),
        jax.ShapeDtypeStruct((N, FE), jnp.float32),
    ],
)

_tc_round = pl.pallas_call(
    _tc_round_body,
    out_shape=jax.ShapeDtypeStruct((N, FE), jnp.float32),
)

_tc_final = pl.pallas_call(
    _tc_final_body,
    out_shape=[
        jax.ShapeDtypeStruct((N, 1), jnp.float32),
        jax.ShapeDtypeStruct((N, 1), jnp.float32),
        jax.ShapeDtypeStruct((N, 1), jnp.float32),
        jax.ShapeDtypeStruct((N, FE), jnp.float32),
    ],
)


def kernel(X_snapshots, edge_index, dt_history, W_in, b_in, W_prop, b_prop,
           Wt1, bt1, Wt2, bt2, W1, b1, W2, b2):
    X = X_snapshots[-1]
    src = edge_index[0].astype(jnp.int32)
    dst = edge_index[1].astype(jnp.int32)
    npad = E_PAD - src.shape[0]
    src_pad = jnp.concatenate(
        [src, jnp.zeros((npad,), jnp.int32)]).reshape(E_PAD // CH, CH)
    dst_pad = jnp.concatenate(
        [dst, jnp.full((npad,), N, jnp.int32)]).reshape(E_PAD // CH, CH)
    eidx = jnp.concatenate(
        [jnp.stack([src_pad, dst_pad], axis=1),
         jnp.zeros((2, 2, CH), jnp.int32)], axis=0)

    deg_parts = _sc_degree(dst_pad)
    isd, G = _tc_proj(deg_parts, X, W_in, b_in.reshape(1, FE))

    for _ in range(2):
        S = _sc_round(eidx, G)
        G = _tc_round(S, isd, W_prop, b_prop.reshape(1, FE))
    S = _sc_round(eidx, G)

    w2p = jnp.pad(W2, ((0, 0), (0, FE - W2.shape[1])))
    b2p = jnp.pad(b2, (0, FE - b2.shape[0])).reshape(1, FE)
    mu, ls, lam, h_last = _tc_final(
        S, isd, W_prop, b_prop.reshape(1, FE),
        dt_history.reshape(-1, 1), Wt1, bt1.reshape(1, -1), Wt2,
        bt2.reshape(1, -1), W1[:FE, :], W1[FE:, :], b1.reshape(1, -1),
        w2p, b2p)
    return mu[:, 0], ls[:, 0], lam[:, 0], h_last


# speedup vs baseline: 1.0613x; 1.0613x over previous
import dataclasses
import functools

import jax
import jax.numpy as jnp
from jax import lax
from jax.experimental import pallas as pl
from jax.experimental.pallas import tpu as pltpu
from jax.experimental.pallas import tpu_sc as plsc

N = 10000
FE = 128
NC = 2
NS = 16
NW = NC * NS
L = 16
CH = 128
CPT = 80
KB = 4
EPT = CH * CPT
E_PAD = EPT * NW
N_PAD = 10112
RPT = N_PAD // NS
ZR = 79

_mesh = plsc.VectorSubcoreMesh(core_axis_name="c", subcore_axis_name="s")

_sc_params = pltpu.CompilerParams()
if "needs_layout_passes" in pltpu.CompilerParams.__dataclass_fields__:
    _sc_params = dataclasses.replace(_sc_params, needs_layout_passes=False)


@functools.partial(
    pl.kernel,
    out_type=jax.ShapeDtypeStruct((NW, N_PAD), jnp.float32),
    mesh=_mesh,
    scratch_types=[
        pltpu.VMEM((N_PAD,), jnp.float32),
        pltpu.VMEM((CPT, CH), jnp.int32),
    ],
    compiler_params=_sc_params,
)
def _sc_degree(dst_hbm, out_hbm, acc_v, idx_v):
    c = lax.axis_index("c")
    s = lax.axis_index("s")
    wid = s * NC + c
    z16 = jnp.zeros((L,), jnp.float32)

    @pl.loop(0, N_PAD, step=L)
    def _(i):
        acc_v[pl.ds(i, L)] = z16

    pltpu.sync_copy(dst_hbm.at[pl.ds(wid * CPT, CPT)], idx_v)
    ones = jnp.ones((L,), jnp.float32)

    @pl.loop(0, CPT)
    def _(r):
        @pl.loop(0, CH, step=L)
        def _(i):
            plsc.addupdate_scatter(acc_v, [idx_v[r, pl.ds(i, L)]], ones)

    pltpu.sync_copy(acc_v, out_hbm.at[wid])


@functools.partial(
    pl.kernel,
    out_type=jax.ShapeDtypeStruct((NC, N_PAD, FE), jnp.float32),
    mesh=_mesh,
    scratch_types=[
        pltpu.VMEM((CPT, 2, CH), jnp.int32),
        pltpu.VMEM((CH, FE), jnp.float32),
        pltpu.VMEM_SHARED((N_PAD, FE), jnp.float32),
    ],
)
def _sc_round(eidx_hbm, g_hbm, out_hbm, idxb, rows, acc_sh):
    c = lax.axis_index("c")
    s = lax.axis_index("s")
    wid = s * NC + c
    base = wid * CPT
    z16 = jnp.zeros((L,), jnp.float32)

    @pl.loop(0, CH)
    def _(r):
        @pl.loop(0, FE, step=L)
        def _(f):
            rows[r, pl.ds(f, L)] = z16

    for k in range(RPT // CH):
        pltpu.sync_copy(rows, acc_sh.at[pl.ds(s * RPT + k * CH, CH)])
    if RPT % CH:
        pltpu.sync_copy(rows.at[pl.ds(0, RPT % CH)],
                        acc_sh.at[pl.ds(s * RPT + (RPT // CH) * CH,
                                        RPT % CH)])

    pltpu.sync_copy(eidx_hbm.at[pl.ds(base, CPT)], idxb)
    plsc.subcore_barrier()

    @pl.loop(0, CPT)
    def _(cc):
        pltpu.sync_copy(g_hbm.at[idxb.at[cc, 0]], rows)
        pltpu.sync_copy(rows, acc_sh.at[idxb.at[cc, 1]], add=True)

    plsc.subcore_barrier()
    pltpu.sync_copy(acc_sh.at[pl.ds(s * RPT, RPT)],
                    out_hbm.at[c, pl.ds(s * RPT, RPT)])


def _tc_proj_body(deg_ref, x_ref, win_ref, bin_ref, isd_ref, g_ref):
    deg = jnp.sum(deg_ref[...][:, :N], axis=0)
    isd = lax.rsqrt(deg + 1.0)
    isd_ref[...] = isd[:, None]
    h = jnp.maximum(
        jnp.dot(x_ref[...], win_ref[...], preferred_element_type=jnp.float32)
        + bin_ref[...], 0.0)
    g_ref[...] = h * isd[:, None]


def _tc_round_body(s_ref, isd_ref, w_ref, b_ref, g_ref):
    isd = isd_ref[...]
    agg = (s_ref[0, :N, :] + s_ref[1, :N, :]) * isd
    h = jnp.maximum(
        jnp.dot(agg, w_ref[...], preferred_element_type=jnp.float32)
        + b_ref[...], 0.0)
    g_ref[...] = h * isd


def _tc_final_body(s_ref, isd_ref, wp_ref, bp_ref, dt_ref, wt1_ref, bt1_ref,
                   wt2_ref, bt2_ref, w1a_ref, w1b_ref, b1_ref, w2_ref, b2_ref,
                   mu_ref, ls_ref, lam_ref, h_ref):
    isd = isd_ref[...]
    agg = (s_ref[0, :N, :] + s_ref[1, :N, :]) * isd
    hl = jnp.maximum(
        jnp.dot(agg, wp_ref[...], preferred_element_type=jnp.float32)
        + bp_ref[...], 0.0)
    h_ref[...] = hl
    e = jnp.maximum(dt_ref[...] * wt1_ref[...] + bt1_ref[...], 0.0)
    me = jnp.mean(e, axis=0, keepdims=True)
    ht = jnp.tanh(
        jnp.dot(me, wt2_ref[...], preferred_element_type=jnp.float32)
        + bt2_ref[...])
    const = jnp.dot(ht, w1b_ref[...], preferred_element_type=jnp.float32) \
        + b1_ref[...]
    hidden = jnp.maximum(
        jnp.dot(hl, w1a_ref[...], preferred_element_type=jnp.float32)
        + const, 0.0)
    out = jnp.dot(hidden, w2_ref[...], preferred_element_type=jnp.float32) \
        + b2_ref[...]
    mu = out[:, 0:1]
    ls2 = out[:, 1:2]
    mu_ref[...] = mu
    ls_ref[...] = ls2
    lam_ref[...] = jnp.exp(mu + 0.5 * jnp.exp(2.0 * ls2))


_tc_proj = pl.pallas_call(
    _tc_proj_body,
    out_shape=[
        jax.ShapeDtypeStruct((N, 1), jnp.float32),
        jax.ShapeDtypeStruct((N, FE), jnp.float32),
    ],
)

_tc_round = pl.pallas_call(
    _tc_round_body,
    out_shape=jax.ShapeDtypeStruct((N, FE), jnp.float32),
)

_tc_final = pl.pallas_call(
    _tc_final_body,
    out_shape=[
        jax.ShapeDtypeStruct((N, 1), jnp.float32),
        jax.ShapeDtypeStruct((N, 1), jnp.float32),
        jax.ShapeDtypeStruct((N, 1), jnp.float32),
        jax.ShapeDtypeStruct((N, FE), jnp.float32),
    ],
)


def kernel(X_snapshots, edge_index, dt_history, W_in, b_in, W_prop, b_prop,
           Wt1, bt1, Wt2, bt2, W1, b1, W2, b2):
    X = X_snapshots[-1]
    src = edge_index[0].astype(jnp.int32)
    dst = edge_index[1].astype(jnp.int32)
    npad = E_PAD - src.shape[0]
    src_pad = jnp.concatenate(
        [src, jnp.zeros((npad,), jnp.int32)]).reshape(E_PAD // CH, CH)
    dst_pad = jnp.concatenate(
        [dst, jnp.full((npad,), N, jnp.int32)]).reshape(E_PAD // CH, CH)
    eidx = jnp.concatenate(
        [jnp.stack([src_pad, dst_pad], axis=1),
         jnp.zeros((2, 2, CH), jnp.int32)], axis=0)

    deg_parts = _sc_degree(dst_pad)
    isd, G = _tc_proj(deg_parts, X, W_in, b_in.reshape(1, FE))

    for _ in range(2):
        S = _sc_round(eidx, G)
        G = _tc_round(S, isd, W_prop, b_prop.reshape(1, FE))
    S = _sc_round(eidx, G)

    w2p = jnp.pad(W2, ((0, 0), (0, FE - W2.shape[1])))
    b2p = jnp.pad(b2, (0, FE - b2.shape[0])).reshape(1, FE)
    mu, ls, lam, h_last = _tc_final(
        S, isd, W_prop, b_prop.reshape(1, FE),
        dt_history.reshape(-1, 1), Wt1, bt1.reshape(1, -1), Wt2,
        bt2.reshape(1, -1), W1[:FE, :], W1[FE:, :], b1.reshape(1, -1),
        w2p, b2p)
    return mu[:, 0], ls[:, 0], lam[:, 0], h_last


# speedup vs baseline: 1.2658x; 1.1927x over previous
import dataclasses
import functools

import jax
import jax.numpy as jnp
from jax import lax
from jax.experimental import pallas as pl
from jax.experimental.pallas import tpu as pltpu
from jax.experimental.pallas import tpu_sc as plsc

N = 10000
FE = 128
NC = 2
NS = 16
NW = NC * NS
L = 16
CH = 128
CPT = 80
EPT = CH * CPT
E_PAD = EPT * NW
N_PAD = 10112

HN = 5056
GB = 5120
ZR = HN
CH2 = 64
NCH = E_PAD // CH2
CPTT = NCH // NS
IB = 4
NP2 = 10112
RP2 = NP2 // NS

_mesh = plsc.VectorSubcoreMesh(core_axis_name="c", subcore_axis_name="s")

_sc_params = pltpu.CompilerParams()
if "needs_layout_passes" in pltpu.CompilerParams.__dataclass_fields__:
    _sc_params = dataclasses.replace(_sc_params, needs_layout_passes=False)


@functools.partial(
    pl.kernel,
    out_type=jax.ShapeDtypeStruct((NW, N_PAD), jnp.float32),
    mesh=_mesh,
    scratch_types=[
        pltpu.VMEM((N_PAD,), jnp.float32),
        pltpu.VMEM((CPT, CH), jnp.int32),
    ],
    compiler_params=_sc_params,
)
def _sc_degree(dst_hbm, out_hbm, acc_v, idx_v):
    c = lax.axis_index("c")
    s = lax.axis_index("s")
    wid = s * NC + c
    z16 = jnp.zeros((L,), jnp.float32)

    @pl.loop(0, N_PAD, step=L)
    def _(i):
        acc_v[pl.ds(i, L)] = z16

    pltpu.sync_copy(dst_hbm.at[pl.ds(wid * CPT, CPT)], idx_v)
    ones = jnp.ones((L,), jnp.float32)

    @pl.loop(0, CPT)
    def _(r):
        @pl.loop(0, CH, step=L)
        def _(i):
            plsc.addupdate_scatter(acc_v, [idx_v[r, pl.ds(i, L)]], ones)

    pltpu.sync_copy(acc_v, out_hbm.at[wid])


@functools.partial(
    pl.kernel,
    out_type=jax.ShapeDtypeStruct((NC, NP2, FE), jnp.float32),
    mesh=_mesh,
    scratch_types=[
        pltpu.VMEM((IB, 2, CH2), jnp.int32),
        pltpu.VMEM((CH2, FE), jnp.float32),
        pltpu.VMEM_SHARED((GB, FE), jnp.float32),
        pltpu.VMEM_SHARED((NP2, FE), jnp.float32),
    ],
)
def _sc_round(eidx_hbm, g_hbm, out_hbm, idxb, rows, g_sh, acc_sh):
    c = lax.axis_index("c")
    s = lax.axis_index("s")
    z16 = jnp.zeros((L,), jnp.float32)

    @pl.loop(0, CH2)
    def _(r):
        @pl.loop(0, FE, step=L)
        def _(f):
            rows[r, pl.ds(f, L)] = z16

    for k in range(RP2 // CH2):
        pltpu.sync_copy(rows, acc_sh.at[pl.ds(s * RP2 + k * CH2, CH2)])
    pltpu.sync_copy(rows.at[pl.ds(0, RP2 % CH2)],
                    acc_sh.at[pl.ds(s * RP2 + (RP2 // CH2) * CH2,
                                    RP2 % CH2)])

    pltpu.sync_copy(g_hbm.at[c, pl.ds(s * (GB // NS), GB // NS)],
                    g_sh.at[pl.ds(s * (GB // NS), GB // NS)])
    plsc.subcore_barrier()

    base = s * CPTT

    @pl.loop(0, CPTT // IB)
    def _(b):
        pltpu.sync_copy(eidx_hbm.at[c].at[pl.ds(base + b * IB, IB)], idxb)
        for k in range(IB):
            pltpu.sync_copy(g_sh.at[idxb.at[k, 0]], rows)
            pltpu.sync_copy(rows, acc_sh.at[idxb.at[k, 1]], add=True)

    plsc.subcore_barrier()
    pltpu.sync_copy(acc_sh.at[pl.ds(s * RP2, RP2)],
                    out_hbm.at[c, pl.ds(s * RP2, RP2)])


def _split_g(g, g_ref):
    g_ref[0, :HN, :] = g[:HN]
    g_ref[0, HN:, :] = jnp.zeros((GB - HN, FE), jnp.float32)
    g_ref[1, :N - HN, :] = g[HN:]
    g_ref[1, N - HN:, :] = jnp.zeros((GB - (N - HN), FE), jnp.float32)


def _tc_proj_body(deg_ref, x_ref, win_ref, bin_ref, isd_ref, g_ref):
    deg = jnp.sum(deg_ref[...][:, :N], axis=0)
    isd = lax.rsqrt(deg + 1.0)
    isd_ref[...] = isd[:, None]
    h = jnp.maximum(
        jnp.dot(x_ref[...], win_ref[...], preferred_element_type=jnp.float32)
        + bin_ref[...], 0.0)
    _split_g(h * isd[:, None], g_ref)


def _tc_round_body(s_ref, isd_ref, w_ref, b_ref, g_ref):
    isd = isd_ref[...]
    agg = (s_ref[0, :N, :] + s_ref[1, :N, :]) * isd
    h = jnp.maximum(
        jnp.dot(agg, w_ref[...], preferred_element_type=jnp.float32)
        + b_ref[...], 0.0)
    _split_g(h * isd, g_ref)


def _tc_final_body(s_ref, isd_ref, wp_ref, bp_ref, dt_ref, wt1_ref, bt1_ref,
                   wt2_ref, bt2_ref, w1a_ref, w1b_ref, b1_ref, w2_ref, b2_ref,
                   mu_ref, ls_ref, lam_ref, h_ref):
    isd = isd_ref[...]
    agg = (s_ref[0, :N, :] + s_ref[1, :N, :]) * isd
    hl = jnp.maximum(
        jnp.dot(agg, wp_ref[...], preferred_element_type=jnp.float32)
        + bp_ref[...], 0.0)
    h_ref[...] = hl
    e = jnp.maximum(dt_ref[...] * wt1_ref[...] + bt1_ref[...], 0.0)
    me = jnp.mean(e, axis=0, keepdims=True)
    ht = jnp.tanh(
        jnp.dot(me, wt2_ref[...], preferred_element_type=jnp.float32)
        + bt2_ref[...])
    const = jnp.dot(ht, w1b_ref[...], preferred_element_type=jnp.float32) \
        + b1_ref[...]
    hidden = jnp.maximum(
        jnp.dot(hl, w1a_ref[...], preferred_element_type=jnp.float32)
        + const, 0.0)
    out = jnp.dot(hidden, w2_ref[...], preferred_element_type=jnp.float32) \
        + b2_ref[...]
    mu = out[:, 0:1]
    ls2 = out[:, 1:2]
    mu_ref[...] = mu
    ls_ref[...] = ls2
    lam_ref[...] = jnp.exp(mu + 0.5 * jnp.exp(2.0 * ls2))


_tc_proj = pl.pallas_call(
    _tc_proj_body,
    out_shape=[
        jax.ShapeDtypeStruct((N, 1), jnp.float32),
        jax.ShapeDtypeStruct((NC, GB, FE), jnp.float32),
    ],
)

_tc_round = pl.pallas_call(
    _tc_round_body,
    out_shape=jax.ShapeDtypeStruct((NC, GB, FE), jnp.float32),
)

_tc_final = pl.pallas_call(
    _tc_final_body,
    out_shape=[
        jax.ShapeDtypeStruct((N, 1), jnp.float32),
        jax.ShapeDtypeStruct((N, 1), jnp.float32),
        jax.ShapeDtypeStruct((N, 1), jnp.float32),
        jax.ShapeDtypeStruct((N, FE), jnp.float32),
    ],
)


def kernel(X_snapshots, edge_index, dt_history, W_in, b_in, W_prop, b_prop,
           Wt1, bt1, Wt2, bt2, W1, b1, W2, b2):
    X = X_snapshots[-1]
    src = edge_index[0].astype(jnp.int32)
    dst = edge_index[1].astype(jnp.int32)
    npad = E_PAD - src.shape[0]
    src_pad = jnp.concatenate([src, jnp.zeros((npad,), jnp.int32)])
    dst_pad = jnp.concatenate([dst, jnp.full((npad,), N, jnp.int32)])
    src0 = jnp.where(src_pad < HN, src_pad, ZR).reshape(NCH, CH2)
    src1 = jnp.where(src_pad >= HN, src_pad - HN, ZR).reshape(NCH, CH2)
    dst_c = dst_pad.reshape(NCH, CH2)
    eidx = jnp.stack([jnp.stack([src0, dst_c], axis=1),
                      jnp.stack([src1, dst_c], axis=1)])

    deg_parts = _sc_degree(dst_pad.reshape(E_PAD // CH, CH))
    isd, G = _tc_proj(deg_parts, X, W_in, b_in.reshape(1, FE))

    for _ in range(2):
        S = _sc_round(eidx, G)
        G = _tc_round(S, isd, W_prop, b_prop.reshape(1, FE))
    S = _sc_round(eidx, G)

    w2p = jnp.pad(W2, ((0, 0), (0, FE - W2.shape[1])))
    b2p = jnp.pad(b2, (0, FE - b2.shape[0])).reshape(1, FE)
    mu, ls, lam, h_last = _tc_final(
        S, isd, W_prop, b_prop.reshape(1, FE),
        dt_history.reshape(-1, 1), Wt1, bt1.reshape(1, -1), Wt2,
        bt2.reshape(1, -1), W1[:FE, :], W1[FE:, :], b1.reshape(1, -1),
        w2p, b2p)
    return mu[:, 0], ls[:, 0], lam[:, 0], h_last
